# baseline (speedup 1.0000x reference)
import jax
import jax.numpy as jnp
from jax.experimental import pallas as pl
from jax.experimental.pallas import tpu as pltpu

_NUM_EXPERTS = 16
_TOP_K = 2
_CHUNK = 1024
_RING = 4
_NSUB = 4
_SUBROWS = _CHUNK // _NSUB


def _chunk_compute(x, w, b, c, scores_ref, wts_ref, idx_ref):
    s = jax.lax.dot_general(
        x, w, (((1,), (1,)), ((), ())),
        preferred_element_type=jnp.float32,
    )
    s = s + b
    m = jnp.max(s, axis=-1, keepdims=True)
    e = jnp.exp(s - m)
    p = e / jnp.sum(e, axis=-1, keepdims=True)

    lane = jax.lax.broadcasted_iota(jnp.int32, p.shape, 1)
    m1 = jnp.max(p, axis=-1, keepdims=True)
    i1 = jnp.min(jnp.where(p == m1, lane, _NUM_EXPERTS), axis=-1, keepdims=True)
    p2 = jnp.where(lane == i1, -jnp.inf, p)
    m2 = jnp.max(p2, axis=-1, keepdims=True)
    i2 = jnp.min(jnp.where(p2 == m2, lane, _NUM_EXPERTS), axis=-1, keepdims=True)

    col = jax.lax.broadcasted_iota(jnp.int32, (p.shape[0], _TOP_K), 1)
    row = pl.ds(c * _CHUNK, _CHUNK)
    scores_ref[row, :] = p
    wts_ref[row, :] = jnp.where(col == 0, m1, m2)
    idx_ref[row, :] = jnp.where(col == 0, i1, i2)


def _router_body(x_hbm, w_ref, b_ref, scores_ref, wts_ref, idx_ref,
                 buf_ref, sem_ref):
    n_chunks = x_hbm.shape[0] // _CHUNK
    w = w_ref[...]
    b = b_ref[...]

    def _copy(c, slot, j):
        return pltpu.make_async_copy(
            x_hbm.at[pl.ds(c * _CHUNK + j * _SUBROWS, _SUBROWS), :],
            buf_ref.at[pl.ds(slot * _CHUNK + j * _SUBROWS, _SUBROWS), :],
            sem_ref.at[slot * _NSUB + j],
        )

    for k in range(_RING):
        for j in range(_NSUB):
            _copy(k, k, j).start()

    def step(c, carry):
        slot = jax.lax.rem(c, _RING)
        for j in range(_NSUB):
            _copy(c, slot, j).wait()
        x = buf_ref[pl.ds(slot * _CHUNK, _CHUNK), :]
        _chunk_compute(x, w, b, c, scores_ref, wts_ref, idx_ref)

        @pl.when(c + _RING < n_chunks)
        def _():
            for j in range(_NSUB):
                _copy(c + _RING, slot, j).start()

        return carry

    jax.lax.fori_loop(0, n_chunks, step, 0, unroll=False)


def kernel(x, W, expert_biases):
    batch_shape = x.shape[:-1]
    d = x.shape[-1]
    flat_x = x.reshape(-1, d)
    n_tok = flat_x.shape[0]
    bias2d = expert_biases.reshape(1, _NUM_EXPERTS)

    scores, wts, idx = pl.pallas_call(
        _router_body,
        in_specs=[
            pl.BlockSpec(memory_space=pltpu.HBM),
            pl.BlockSpec(memory_space=pltpu.VMEM),
            pl.BlockSpec(memory_space=pltpu.VMEM),
        ],
        out_specs=[
            pl.BlockSpec(memory_space=pltpu.VMEM),
            pl.BlockSpec(memory_space=pltpu.VMEM),
            pl.BlockSpec(memory_space=pltpu.VMEM),
        ],
        out_shape=[
            jax.ShapeDtypeStruct((n_tok, _NUM_EXPERTS), jnp.float32),
            jax.ShapeDtypeStruct((n_tok, _TOP_K), jnp.float32),
            jax.ShapeDtypeStruct((n_tok, _TOP_K), jnp.int32),
        ],
        scratch_shapes=[
            pltpu.VMEM((_RING * _CHUNK, d), jnp.float32),
            pltpu.SemaphoreType.DMA((_RING * _NSUB,)),
        ],
    )(flat_x, W, bias2d)

    return (
        scores.reshape(*batch_shape, _NUM_EXPERTS),
        wts.reshape(*batch_shape, _TOP_K),
        idx.reshape(*batch_shape, _TOP_K),
    )


# speedup vs baseline: 1.5500x; 1.5500x over previous
import jax
import jax.numpy as jnp
from jax.experimental import pallas as pl
from jax.experimental.pallas import tpu as pltpu

_NUM_EXPERTS = 16
_TOP_K = 2
_CHUNK = 1024
_RING = 4
_NSUB = 4
_SUBROWS = _CHUNK // _NSUB


def _chunk_compute(x, w, b, c, seq, scores_ref, wts_ref, idx_ref):
    s = jax.lax.dot_general(
        w, x, (((1,), (1,)), ((), ())),
        preferred_element_type=jnp.float32,
    )
    s = s + b
    m = jnp.max(s, axis=0, keepdims=True)
    e = jnp.exp(s - m)
    p = e / jnp.sum(e, axis=0, keepdims=True)

    erow = jax.lax.broadcasted_iota(jnp.int32, p.shape, 0)
    m1 = jnp.max(p, axis=0, keepdims=True)
    i1 = jnp.min(jnp.where(p == m1, erow, _NUM_EXPERTS), axis=0, keepdims=True)
    p2 = jnp.where(erow == i1, -jnp.inf, p)
    m2 = jnp.max(p2, axis=0, keepdims=True)
    i2 = jnp.min(jnp.where(p2 == m2, erow, _NUM_EXPERTS), axis=0, keepdims=True)

    batch = c // (seq // _CHUNK)
    off = (c % (seq // _CHUNK)) * _CHUNK
    col = pl.ds(off, _CHUNK)
    scores_ref[batch, :, col] = p
    wts_ref[batch, :, col] = jnp.concatenate([m1, m2], 0)
    idx_ref[batch, :, col] = jnp.concatenate([i1, i2], 0)


def _make_router_body(seq):
    def _router_body(x_hbm, w_ref, b_ref, scores_ref, wts_ref, idx_ref,
                     buf_ref, sem_ref):
        n_chunks = x_hbm.shape[0] // _CHUNK
        w = w_ref[...]
        b = b_ref[...]

        def _copy(c, slot, j):
            return pltpu.make_async_copy(
                x_hbm.at[pl.ds(c * _CHUNK + j * _SUBROWS, _SUBROWS), :],
                buf_ref.at[pl.ds(slot * _CHUNK + j * _SUBROWS, _SUBROWS), :],
                sem_ref.at[slot * _NSUB + j],
            )

        for k in range(_RING):
            for j in range(_NSUB):
                _copy(k, k, j).start()

        def step(c, carry):
            slot = jax.lax.rem(c, _RING)
            for j in range(_NSUB):
                _copy(c, slot, j).wait()
            x = buf_ref[pl.ds(slot * _CHUNK, _CHUNK), :]
            _chunk_compute(x, w, b, c, seq, scores_ref, wts_ref, idx_ref)

            @pl.when(c + _RING < n_chunks)
            def _():
                for j in range(_NSUB):
                    _copy(c + _RING, slot, j).start()

            return carry

        jax.lax.fori_loop(0, n_chunks, step, 0, unroll=False)

    return _router_body


def kernel(x, W, expert_biases):
    batch_shape = x.shape[:-1]
    d = x.shape[-1]
    flat_x = x.reshape(-1, d)
    n_tok = flat_x.shape[0]
    seq = batch_shape[-1] if batch_shape else 1
    nb = n_tok // seq
    bias_col = expert_biases.reshape(_NUM_EXPERTS, 1)

    scores_t, wts_t, idx_t = pl.pallas_call(
        _make_router_body(seq),
        in_specs=[
            pl.BlockSpec(memory_space=pltpu.HBM),
            pl.BlockSpec(memory_space=pltpu.VMEM),
            pl.BlockSpec(memory_space=pltpu.VMEM),
        ],
        out_specs=[
            pl.BlockSpec(memory_space=pltpu.VMEM),
            pl.BlockSpec(memory_space=pltpu.VMEM),
            pl.BlockSpec(memory_space=pltpu.VMEM),
        ],
        out_shape=[
            jax.ShapeDtypeStruct((nb, _NUM_EXPERTS, seq), jnp.float32),
            jax.ShapeDtypeStruct((nb, _TOP_K, seq), jnp.float32),
            jax.ShapeDtypeStruct((nb, _TOP_K, seq), jnp.int32),
        ],
        scratch_shapes=[
            pltpu.VMEM((_RING * _CHUNK, d), jnp.float32),
            pltpu.SemaphoreType.DMA((_RING * _NSUB,)),
        ],
    )(flat_x, W, bias_col)

    scores = jnp.swapaxes(scores_t, 1, 2)
    wts = jnp.swapaxes(wts_t, 1, 2)
    idx = jnp.swapaxes(idx_t, 1, 2)
    return (
        scores.reshape(*batch_shape, _NUM_EXPERTS),
        wts.reshape(*batch_shape, _TOP_K),
        idx.reshape(*batch_shape, _TOP_K),
    )


# speedup vs baseline: 1.6226x; 1.0468x over previous
import jax
import jax.numpy as jnp
from jax.experimental import pallas as pl
from jax.experimental.pallas import tpu as pltpu

_NUM_EXPERTS = 16
_TOP_K = 2
_CHUNK = 1024
_RING = 6
_NSUB = 4
_SUBROWS = _CHUNK // _NSUB


def _chunk_compute(x, w, b, c, seq, scores_ref, wts_ref, idx_ref):
    s = jax.lax.dot_general(
        w, x, (((1,), (1,)), ((), ())),
        preferred_element_type=jnp.float32,
    )
    s = s + b
    m = jnp.max(s, axis=0, keepdims=True)
    e = jnp.exp(s - m)
    p = e / jnp.sum(e, axis=0, keepdims=True)

    erow = jax.lax.broadcasted_iota(jnp.int32, p.shape, 0)
    m1 = jnp.max(p, axis=0, keepdims=True)
    i1 = jnp.min(jnp.where(p == m1, erow, _NUM_EXPERTS), axis=0, keepdims=True)
    p2 = jnp.where(erow == i1, -jnp.inf, p)
    m2 = jnp.max(p2, axis=0, keepdims=True)
    i2 = jnp.min(jnp.where(p2 == m2, erow, _NUM_EXPERTS), axis=0, keepdims=True)

    batch = c // (seq // _CHUNK)
    off = (c % (seq // _CHUNK)) * _CHUNK
    col = pl.ds(off, _CHUNK)
    scores_ref[batch, :, col] = p
    wts_ref[batch, :, col] = jnp.concatenate([m1, m2], 0)
    idx_ref[batch, :, col] = jnp.concatenate([i1, i2], 0)


def _make_router_body(seq):
    def _router_body(x_hbm, w_ref, b_ref, scores_ref, wts_ref, idx_ref,
                     buf_ref, sem_ref):
        n_chunks = x_hbm.shape[0] // _CHUNK
        w = w_ref[...]
        b = jnp.swapaxes(b_ref[...], 0, 1)

        def _copy(c, slot, j):
            return pltpu.make_async_copy(
                x_hbm.at[pl.ds(c * _CHUNK + j * _SUBROWS, _SUBROWS), :],
                buf_ref.at[pl.ds(slot * _CHUNK + j * _SUBROWS, _SUBROWS), :],
                sem_ref.at[slot * _NSUB + j],
            )

        for k in range(_RING):
            for j in range(_NSUB):
                _copy(k, k, j).start()

        def step(c, carry):
            slot = jax.lax.rem(c, _RING)
            for j in range(_NSUB):
                _copy(c, slot, j).wait()
            x = buf_ref[pl.ds(slot * _CHUNK, _CHUNK), :]
            _chunk_compute(x, w, b, c, seq, scores_ref, wts_ref, idx_ref)

            @pl.when(c + _RING < n_chunks)
            def _():
                for j in range(_NSUB):
                    _copy(c + _RING, slot, j).start()

            return carry

        jax.lax.fori_loop(0, n_chunks, step, 0, unroll=False)

    return _router_body


def kernel(x, W, expert_biases):
    batch_shape = x.shape[:-1]
    d = x.shape[-1]
    flat_x = x.reshape(-1, d)
    n_tok = flat_x.shape[0]
    seq = batch_shape[-1] if batch_shape else 1
    nb = n_tok // seq
    bias_row = expert_biases.reshape(1, _NUM_EXPERTS)

    scores_t, wts_t, idx_t = pl.pallas_call(
        _make_router_body(seq),
        in_specs=[
            pl.BlockSpec(memory_space=pltpu.HBM),
            pl.BlockSpec(memory_space=pltpu.VMEM),
            pl.BlockSpec(memory_space=pltpu.VMEM),
        ],
        out_specs=[
            pl.BlockSpec(memory_space=pltpu.VMEM),
            pl.BlockSpec(memory_space=pltpu.VMEM),
            pl.BlockSpec(memory_space=pltpu.VMEM),
        ],
        out_shape=[
            jax.ShapeDtypeStruct((nb, _NUM_EXPERTS, seq), jnp.float32),
            jax.ShapeDtypeStruct((nb, _TOP_K, seq), jnp.float32),
            jax.ShapeDtypeStruct((nb, _TOP_K, seq), jnp.int32),
        ],
        scratch_shapes=[
            pltpu.VMEM((_RING * _CHUNK, d), jnp.float32),
            pltpu.SemaphoreType.DMA((_RING * _NSUB,)),
        ],
    )(flat_x, W, bias_row)

    scores = jnp.swapaxes(scores_t, 1, 2)
    wts = jnp.swapaxes(wts_t, 1, 2)
    idx = jnp.swapaxes(idx_t, 1, 2)
    return (
        scores.reshape(*batch_shape, _NUM_EXPERTS),
        wts.reshape(*batch_shape, _TOP_K),
        idx.reshape(*batch_shape, _TOP_K),
    )
